# baseline (speedup 1.0000x reference)
import functools

import jax
import jax.numpy as jnp
from jax import lax
from jax.experimental import pallas as pl
from jax.experimental.pallas import tpu as pltpu
from jax.experimental.pallas import tpu_sc as plsc

VOCAB = 1000000
B = 4096
T = 200
D = 64
N = B * T
NC = 2
NS = 16
NW = NC * NS
ROWS_W = N // NW
CHUNK = 128
N_CHUNKS = ROWS_W // CHUNK
NB = 2
NST = 4
NSLOT = NST * NB
NGROUP = N_CHUNKS // NB
POS_BUF = T + CHUNK


@jax.jit
def _embed(tok2, idx3, pos2):
    mesh = plsc.VectorSubcoreMesh(core_axis_name="c", subcore_axis_name="s")

    @functools.partial(
        pl.kernel,
        out_type=jax.ShapeDtypeStruct((N, 2 * D), jnp.float32),
        mesh=mesh,
        scratch_types=[
            pltpu.VMEM((N_CHUNKS, CHUNK), jnp.int32),
            pltpu.VMEM((NSLOT, CHUNK, D), jnp.float32),
            pltpu.SemaphoreType.DMA((NSLOT,)),
            pltpu.SemaphoreType.DMA((NSLOT,)),
            pltpu.SemaphoreType.DMA((NSLOT,)),
        ],
        compiler_params=pltpu.CompilerParams(use_tc_tiling_on_sc=False),
    )
    def k(tok_hbm, idx_hbm, pos2_hbm, out_hbm, idx_v, bufs, semp, semg, sems):
        wid = lax.axis_index("s") * NC + lax.axis_index("c")
        base = wid * ROWS_W
        pltpu.sync_copy(idx_hbm.at[wid], idx_v)

        def prefill(g, sp, b):
            j = g * NB + b
            rem = lax.rem(j * CHUNK, T)
            s = sp * NB + b
            pltpu.async_copy(pos2_hbm.at[pl.ds(wid * POS_BUF + rem, CHUNK)],
                             bufs.at[s], semp.at[s])

        def wait_prefill(sp, b):
            s = sp * NB + b
            pltpu.make_async_copy(pos2_hbm.at[pl.ds(0, CHUNK)], bufs.at[s],
                                  semp.at[s]).wait()

        def gather(g, sp, b):
            s = sp * NB + b
            pltpu.async_copy(tok_hbm.at[idx_v.at[g * NB + b]], bufs.at[s],
                             semg.at[s], add=True)

        def wait_gather(g, sp, b):
            s = sp * NB + b
            pltpu.make_async_copy(tok_hbm.at[idx_v.at[g * NB + b]],
                                  bufs.at[s], semg.at[s]).wait()

        def store(g, sp, b):
            j = g * NB + b
            s = sp * NB + b
            pltpu.async_copy(bufs.at[s],
                             out_hbm.at[pl.ds(base + j * CHUNK, CHUNK),
                                        pl.ds(0, D)], sems.at[s])

        def wait_store(sp, b):
            s = sp * NB + b
            pltpu.make_async_copy(pos2_hbm.at[pl.ds(0, CHUNK)], bufs.at[s],
                                  sems.at[s]).wait()

        for b in range(NB):
            prefill(0, 0, b)
        for b in range(NB):
            prefill(1, 1, b)
        for b in range(NB):
            wait_prefill(0, b)
            gather(0, 0, b)

        @pl.loop(0, (NGROUP + NST - 1) // NST)
        def _(g3):
            for p in range(NST):
                g = g3 * NST + p

                @pl.when(g + 2 < NGROUP)
                def _():
                    sp2 = (p + 2) % NST
                    for b in range(NB):
                        @pl.when(g >= NST - 2)
                        def _():
                            wait_store(sp2, b)
                        prefill(g + 2, sp2, b)

                @pl.when(g + 1 < NGROUP)
                def _():
                    sp1 = (p + 1) % NST
                    for b in range(NB):
                        wait_prefill(sp1, b)
                        gather(g + 1, sp1, b)

                @pl.when(g < NGROUP)
                def _():
                    for b in range(NB):
                        wait_gather(g, p, b)
                        store(g, p, b)

        for sp in range(NST):
            for b in range(NB):
                wait_store(sp, b)

    return k(tok2, idx3, pos2)


def kernel(x_ids, tok_weight, pos_weight):
    idx3 = (x_ids.astype(jnp.int32) * 2).reshape(NW, N_CHUNKS, CHUNK)
    tok2 = jnp.pad(tok_weight, ((0, 0), (0, D))).reshape(2 * VOCAB, D)
    pos2 = jnp.tile(jnp.concatenate([pos_weight[:T], pos_weight[:CHUNK]], 0),
                    (NW, 1))
    out = _embed(tok2, idx3, pos2)
    return out[:, :D].reshape(B, T, D)


# speedup vs baseline: 1.1013x; 1.1013x over previous
import functools

import jax
import jax.numpy as jnp
from jax import lax
from jax.experimental import pallas as pl
from jax.experimental.pallas import tpu as pltpu
from jax.experimental.pallas import tpu_sc as plsc

VOCAB = 1000000
B = 4096
T = 200
D = 64
N = B * T
NC = 2
NS = 16
NW = NC * NS
ROWS_W = N // NW
CHUNK = 128
N_CHUNKS = ROWS_W // CHUNK
NB = 2
NST = 3
NSLOT = NST * NB
NGROUP = N_CHUNKS // NB
POS_BUF = T + CHUNK


@jax.jit
def _embed(tok2, idx3, pos2):
    mesh = plsc.VectorSubcoreMesh(core_axis_name="c", subcore_axis_name="s")

    @functools.partial(
        pl.kernel,
        out_type=jax.ShapeDtypeStruct((N, 2 * D), jnp.float32),
        mesh=mesh,
        scratch_types=[
            pltpu.VMEM((N_CHUNKS, CHUNK), jnp.int32),
            pltpu.VMEM((NSLOT, CHUNK, D), jnp.float32),
            pltpu.SemaphoreType.DMA((NSLOT,)),
            pltpu.SemaphoreType.DMA((NSLOT,)),
            pltpu.SemaphoreType.DMA((NSLOT,)),
            pltpu.VMEM_SHARED((POS_BUF, D), jnp.float32),
        ],
        compiler_params=pltpu.CompilerParams(use_tc_tiling_on_sc=False),
    )
    def k(tok_hbm, idx_hbm, pos2_hbm, out_hbm, idx_v, bufs, semp, semg,
          sems, pos_sh):
        wid = lax.axis_index("s") * NC + lax.axis_index("c")
        base = wid * ROWS_W
        @pl.when(lax.axis_index("s") == 0)
        def _():
            pltpu.sync_copy(pos2_hbm.at[pl.ds(0, POS_BUF)], pos_sh)
        pltpu.sync_copy(idx_hbm.at[wid], idx_v)
        plsc.subcore_barrier()

        def prefill(g, sp, b):
            j = g * NB + b
            rem = lax.rem(j * CHUNK, T)
            s = sp * NB + b
            pltpu.async_copy(pos_sh.at[pl.ds(rem, CHUNK)], bufs.at[s],
                             semp.at[s])

        def wait_prefill(sp, b):
            s = sp * NB + b
            pltpu.make_async_copy(pos_sh.at[pl.ds(0, CHUNK)], bufs.at[s],
                                  semp.at[s]).wait()

        def gather(g, sp, b):
            s = sp * NB + b
            pltpu.async_copy(tok_hbm.at[idx_v.at[g * NB + b]], bufs.at[s],
                             semg.at[s], add=True)

        def wait_gather(g, sp, b):
            s = sp * NB + b
            pltpu.make_async_copy(tok_hbm.at[idx_v.at[g * NB + b]],
                                  bufs.at[s], semg.at[s]).wait()

        def store(g, sp, b):
            j = g * NB + b
            s = sp * NB + b
            pltpu.async_copy(bufs.at[s],
                             out_hbm.at[pl.ds(base + j * CHUNK, CHUNK),
                                        pl.ds(0, D)], sems.at[s])

        def wait_store(sp, b):
            s = sp * NB + b
            pltpu.make_async_copy(pos2_hbm.at[pl.ds(0, CHUNK)], bufs.at[s],
                                  sems.at[s]).wait()

        for b in range(NB):
            prefill(0, 0, b)
        for b in range(NB):
            prefill(1, 1, b)
        for b in range(NB):
            wait_prefill(0, b)
            gather(0, 0, b)

        @pl.loop(0, (NGROUP + NST - 1) // NST)
        def _(g3):
            for p in range(NST):
                g = g3 * NST + p

                @pl.when(g + 2 < NGROUP)
                def _():
                    sp2 = (p + 2) % NST
                    for b in range(NB):
                        @pl.when(g >= 1)
                        def _():
                            wait_store(sp2, b)
                        prefill(g + 2, sp2, b)

                @pl.when(g + 1 < NGROUP)
                def _():
                    sp1 = (p + 1) % NST
                    for b in range(NB):
                        wait_prefill(sp1, b)
                        gather(g + 1, sp1, b)

                @pl.when(g < NGROUP)
                def _():
                    for b in range(NB):
                        wait_gather(g, p, b)
                        store(g, p, b)

        for sp in range(NST):
            for b in range(NB):
                wait_store(sp, b)

    return k(tok2, idx3, pos2)


def kernel(x_ids, tok_weight, pos_weight):
    idx3 = (x_ids.astype(jnp.int32) * 2).reshape(NW, N_CHUNKS, CHUNK)
    tok2 = jnp.pad(tok_weight, ((0, 0), (0, D))).reshape(2 * VOCAB, D)
    pos2 = jnp.tile(jnp.concatenate([pos_weight[:T], pos_weight[:CHUNK]], 0),
                    (NW, 1))
    out = _embed(tok2, idx3, pos2)
    return out[:, :D].reshape(B, T, D)
